# baseline (speedup 1.0000x reference)
import functools

import jax
import jax.numpy as jnp
from jax import lax
from jax.experimental import pallas as pl
from jax.experimental.pallas import tpu as pltpu
from jax.experimental.pallas import tpu_sc as plsc

_VOCAB = 1000000
_DIM = 64
_BATCH = 4096
_HIST = 50
_NW = 32
_HB = _BATCH // _NW
_PER_W = _HIST * _HB
_NBUF = 5


def _make_sc_kernel():
    mesh = plsc.VectorSubcoreMesh(core_axis_name="c", subcore_axis_name="s")

    @functools.partial(
        pl.kernel,
        mesh=mesh,
        compiler_params=pltpu.CompilerParams(
            use_tc_tiling_on_sc=False, needs_layout_passes=False),
        out_type=jax.ShapeDtypeStruct((_HIST, _DIM, _BATCH), jnp.float32),
        scratch_types=[
            pltpu.VMEM((_PER_W,), jnp.int32),
            pltpu.VMEM((_PER_W,), jnp.int32),
        ] + [pltpu.VMEM((_HB, _DIM), jnp.float32)] * _NBUF
          + [pltpu.VMEM((_DIM, _HB), jnp.float32)] * _NBUF
          + [
            pltpu.SemaphoreType.DMA,
            pltpu.SemaphoreType.DMA,
        ] + [pltpu.SemaphoreType.DMA] * _NBUF
          + [pltpu.SemaphoreType.DMA] * _NBUF,
    )
    def tok_emb(xt_hbm, table_hbm, idmap_hbm, out_hbm,
                xv, mv, *bufs):
        rows = bufs[:_NBUF]
        tbuf = bufs[_NBUF:2 * _NBUF]
        sem_x = bufs[2 * _NBUF]
        sem_m = bufs[2 * _NBUF + 1]
        sem_r = bufs[2 * _NBUF + 2:2 * _NBUF + 2 + _NBUF]
        sem_o = bufs[2 * _NBUF + 2 + _NBUF:]
        wid = lax.axis_index("s") * 2 + lax.axis_index("c")
        col0 = wid * _HB
        iota = lax.iota(jnp.int32, 16)

        def g_copy(h, s):
            return pltpu.make_async_copy(
                table_hbm.at[mv.at[pl.ds(h * _HB, _HB)]], rows[s],
                sem_r[s])

        def o_copy(h, s):
            return pltpu.make_async_copy(
                tbuf[s], out_hbm.at[h, :, pl.ds(col0, _HB)], sem_o[s])

        def transpose(s):
            for c in range(_HB):
                c_vec = jnp.full((16,), c, jnp.int32)
                for g in range(_DIM // 16):
                    vec = rows[s][c, pl.ds(g * 16, 16)]
                    plsc.store_scatter(tbuf[s], [iota + (g * 16), c_vec], vec)

        for h in range(_HIST):
            pltpu.make_async_copy(
                xt_hbm.at[h, pl.ds(col0, _HB)],
                xv.at[pl.ds(h * _HB, _HB)], sem_x).start()
        for h in range(_HIST):
            pltpu.make_async_copy(
                xt_hbm.at[h, pl.ds(col0, _HB)],
                xv.at[pl.ds(h * _HB, _HB)], sem_x).wait()
        pltpu.make_async_copy(idmap_hbm.at[xv], mv, sem_m).start()
        pltpu.make_async_copy(idmap_hbm.at[xv], mv, sem_m).wait()

        def outer(o, carry):
            h0 = o * _NBUF
            for i in range(_NBUF):
                s = i
                h = h0 + i

                @pl.when(o >= 1)
                def _():
                    o_copy(h - _NBUF, s).wait()
                g_copy(h, s).start()

                s1 = (i - 1) % _NBUF
                if i == 0:
                    @pl.when(o >= 1)
                    def _():
                        g_copy(h - 1, s1).wait()
                        transpose(s1)
                        o_copy(h - 1, s1).start()
                else:
                    g_copy(h - 1, s1).wait()
                    transpose(s1)
                    o_copy(h - 1, s1).start()
            return carry

        lax.fori_loop(0, _HIST // _NBUF, outer, 0)

        g_copy(_HIST - 1, _NBUF - 1).wait()
        transpose(_NBUF - 1)
        o_copy(_HIST - 1, _NBUF - 1).start()
        for i in range(_NBUF):
            o_copy(_HIST - _NBUF + i, i).wait()

    return tok_emb


_SC_KERNEL = _make_sc_kernel()


def kernel(x, table, id_mapper):
    xt = jnp.transpose(x)
    out = _SC_KERNEL(xt, table, id_mapper)
    return jnp.transpose(out, (2, 0, 1))


# speedup vs baseline: 1.1985x; 1.1985x over previous
import functools

import jax
import jax.numpy as jnp
from jax import lax
from jax.experimental import pallas as pl
from jax.experimental.pallas import tpu as pltpu
from jax.experimental.pallas import tpu_sc as plsc

_VOCAB = 1000000
_DIM = 64
_BATCH = 4096
_HIST = 50
_N = _BATCH * _HIST
_NW = 32
_HB = _BATCH // _NW
_PER_W = _HIST * _HB
_NBUF = 8
_SKEW = 4


def _flatten_body(xt_ref, out_ref):
    out_ref[...] = xt_ref[...].reshape(_N // 128, 128)


def _flatten_x(xt):
    return pl.pallas_call(
        _flatten_body,
        in_specs=[pl.BlockSpec((_HIST, _BATCH), lambda: (0, 0))],
        out_specs=pl.BlockSpec((_N // 128, 128), lambda: (0, 0)),
        out_shape=jax.ShapeDtypeStruct((_N // 128, 128), jnp.int32),
    )(xt)


def _make_sc_kernel():
    mesh = plsc.VectorSubcoreMesh(core_axis_name="c", subcore_axis_name="s")

    @functools.partial(
        pl.kernel,
        mesh=mesh,
        compiler_params=pltpu.CompilerParams(
            use_tc_tiling_on_sc=False, needs_layout_passes=False),
        out_type=jax.ShapeDtypeStruct((_HIST, _BATCH, _DIM), jnp.float32),
        scratch_types=[
            pltpu.VMEM((_PER_W,), jnp.int32),
            pltpu.VMEM((_PER_W,), jnp.int32),
        ] + [pltpu.VMEM((_HB, _DIM), jnp.float32)] * _NBUF
          + [
            pltpu.SemaphoreType.DMA,
            pltpu.SemaphoreType.DMA,
        ] + [pltpu.SemaphoreType.DMA] * _NBUF
          + [pltpu.SemaphoreType.DMA] * _NBUF,
    )
    def tok_emb(xf_hbm, table_hbm, idmap_hbm, out_hbm, xv, mv, *bufs):
        rows = bufs[:_NBUF]
        sem_x = bufs[_NBUF]
        sem_m = bufs[_NBUF + 1]
        sem_r = bufs[_NBUF + 2:2 * _NBUF + 2]
        sem_o = bufs[2 * _NBUF + 2:]
        wid = lax.axis_index("s") * 2 + lax.axis_index("c")
        col0 = wid * _HB

        def x_copy(h):
            return pltpu.make_async_copy(
                xf_hbm.at[h * _NW + wid],
                xv.at[pl.ds(h * _HB, _HB)], sem_x)

        def g_copy(h, s):
            return pltpu.make_async_copy(
                table_hbm.at[mv.at[pl.ds(h * _HB, _HB)]], rows[s], sem_r[s])

        def o_copy(h, s):
            return pltpu.make_async_copy(
                rows[s], out_hbm.at[h, pl.ds(col0, _HB), :], sem_o[s])

        for h in range(_HIST):
            x_copy(h).start()
        for h in range(_HIST):
            x_copy(h).wait()
        pltpu.make_async_copy(idmap_hbm.at[xv], mv, sem_m).start()
        pltpu.make_async_copy(idmap_hbm.at[xv], mv, sem_m).wait()

        for h in range(_HIST + _SKEW):
            if h < _HIST:
                s = h % _NBUF
                if h >= _NBUF:
                    o_copy(h - _NBUF, s).wait()
                g_copy(h, s).start()
            j = h - _SKEW
            if 0 <= j < _HIST:
                sj = j % _NBUF
                g_copy(j, sj).wait()
                o_copy(j, sj).start()

        for j in range(_HIST - _NBUF, _HIST):
            o_copy(j, j % _NBUF).wait()

    return tok_emb


_SC_KERNEL = _make_sc_kernel()


def kernel(x, table, id_mapper):
    xf = _flatten_x(jnp.transpose(x))
    out = _SC_KERNEL(xf, table, id_mapper)
    return jnp.transpose(out, (1, 0, 2))


# speedup vs baseline: 1.6392x; 1.3677x over previous
import functools

import jax
import jax.numpy as jnp
from jax import lax
from jax.experimental import pallas as pl
from jax.experimental.pallas import tpu as pltpu
from jax.experimental.pallas import tpu_sc as plsc

_VOCAB = 1000000
_DIM = 64
_BATCH = 4096
_HIST = 50
_N = _BATCH * _HIST
_NW = 32
_HB = _BATCH // _NW
_PER_W = _HIST * _HB
_NBUF = 8
_SKEW = 4


def _flatten_body(xt_ref, out_ref):
    out_ref[...] = xt_ref[...].reshape(_N)


def _flatten_x(xt):
    return pl.pallas_call(
        _flatten_body,
        in_specs=[pl.BlockSpec((_HIST, _BATCH), lambda: (0, 0))],
        out_specs=pl.BlockSpec((_N,), lambda: (0,)),
        out_shape=jax.ShapeDtypeStruct((_N,), jnp.int32),
    )(xt)


_TBLK = 8192
_TGRID = -(-_VOCAB // _TBLK)
_TROWS = _TGRID * _TBLK


def _tflat_body(tt_ref, out_ref):
    t = tt_ref[...].T
    t3 = t.reshape(_TBLK // 2, 2, _DIM)
    out_ref[:, 0:_DIM] = t3[:, 0, :]
    out_ref[:, _DIM:2 * _DIM] = t3[:, 1, :]


def _flatten_table(tt):
    return pl.pallas_call(
        _tflat_body,
        grid=(_TGRID,),
        in_specs=[pl.BlockSpec((_DIM, _TBLK), lambda i: (0, i))],
        out_specs=pl.BlockSpec((_TBLK // 2, 2 * _DIM), lambda i: (i, 0)),
        out_shape=jax.ShapeDtypeStruct((_TROWS // 2, 2 * _DIM), jnp.float32),
    )(tt)


def _make_sc_kernel():
    mesh = plsc.VectorSubcoreMesh(core_axis_name="c", subcore_axis_name="s")

    @functools.partial(
        pl.kernel,
        mesh=mesh,
        compiler_params=pltpu.CompilerParams(
            use_tc_tiling_on_sc=False, needs_layout_passes=False),
        out_type=jax.ShapeDtypeStruct((_HIST, _BATCH, _DIM), jnp.float32),
        scratch_types=[
            pltpu.VMEM((_PER_W,), jnp.int32),
            pltpu.VMEM((_PER_W,), jnp.int32),
        ] + [pltpu.VMEM((_HB, _DIM), jnp.float32)] * _NBUF
          + [
            pltpu.SemaphoreType.DMA,
            pltpu.SemaphoreType.DMA,
        ] + [pltpu.SemaphoreType.DMA] * _NBUF
          + [pltpu.SemaphoreType.DMA] * _NBUF,
    )
    def tok_emb(xf_hbm, table_hbm, idmap_hbm, out_hbm, xv, mv, *bufs):
        rows = bufs[:_NBUF]
        sem_x = bufs[_NBUF]
        sem_m = bufs[_NBUF + 1]
        sem_r = bufs[_NBUF + 2:2 * _NBUF + 2]
        sem_o = bufs[2 * _NBUF + 2:]
        wid = lax.axis_index("s") * 2 + lax.axis_index("c")
        col0 = wid * _HB

        def x_copy(h):
            return pltpu.make_async_copy(
                xf_hbm.at[pl.ds(h * _BATCH + col0, _HB)],
                xv.at[pl.ds(h * _HB, _HB)], sem_x)

        def g_copy(h, s):
            return pltpu.make_async_copy(
                table_hbm.at[mv.at[pl.ds(h * _HB, _HB)]], rows[s], sem_r[s])

        def o_copy(h, s):
            return pltpu.make_async_copy(
                rows[s], out_hbm.at[h, pl.ds(col0, _HB), :], sem_o[s])

        for h in range(_HIST):
            x_copy(h).start()
        for h in range(_HIST):
            x_copy(h).wait()
        pltpu.make_async_copy(idmap_hbm.at[xv], mv, sem_m).start()
        pltpu.make_async_copy(idmap_hbm.at[xv], mv, sem_m).wait()

        for h in range(_HIST + _SKEW):
            if h < _HIST:
                s = h % _NBUF
                if h >= _NBUF:
                    o_copy(h - _NBUF, s).wait()
                g_copy(h, s).start()
            j = h - _SKEW
            if 0 <= j < _HIST:
                sj = j % _NBUF
                g_copy(j, sj).wait()
                o_copy(j, sj).start()

        for j in range(_HIST - _NBUF, _HIST):
            o_copy(j, j % _NBUF).wait()

    return tok_emb


_SC_KERNEL = _make_sc_kernel()


def kernel(x, table, id_mapper):
    xf = _flatten_x(jnp.transpose(x))
    tf = _flatten_table(jnp.transpose(table))
    table2 = tf.reshape(_TROWS, _DIM)
    out = _SC_KERNEL(xf, table2, id_mapper)
    return jnp.transpose(out, (1, 0, 2))


# speedup vs baseline: 2.4815x; 1.5139x over previous
import functools

import jax
import jax.numpy as jnp
from jax import lax
from jax.experimental import pallas as pl
from jax.experimental.pallas import tpu as pltpu
from jax.experimental.pallas import tpu_sc as plsc

_VOCAB = 1000000
_DIM = 64
_BATCH = 4096
_HIST = 50
_N = _BATCH * _HIST
_NW = 32
_HB = _BATCH // _NW
_PER_W = _HIST * _HB
_NBUF = 8
_SKEW = 4


def _flatten_body(xt_ref, out_ref):
    out_ref[...] = xt_ref[...].reshape(_N)


def _flatten_x(xt):
    return pl.pallas_call(
        _flatten_body,
        in_specs=[pl.BlockSpec((_HIST, _BATCH), lambda: (0, 0))],
        out_specs=pl.BlockSpec((_N,), lambda: (0,)),
        out_shape=jax.ShapeDtypeStruct((_N,), jnp.int32),
    )(xt)


_TBLK = 8192
_TBLK_HALF_LOG2 = 12
_TGRID = -(-_VOCAB // _TBLK)
_TROWS = _TGRID * _TBLK


def _tflat_body(tt_ref, out_ref):
    t = tt_ref[...]
    v = jnp.concatenate([t[:, :_TBLK // 2], t[:, _TBLK // 2:]], axis=0)
    out_ref[...] = v.T


def _flatten_table(tt):
    return pl.pallas_call(
        _tflat_body,
        grid=(_TGRID,),
        in_specs=[pl.BlockSpec((_DIM, _TBLK), lambda i: (0, i))],
        out_specs=pl.BlockSpec((_TBLK // 2, 2 * _DIM), lambda i: (i, 0)),
        out_shape=jax.ShapeDtypeStruct((_TROWS // 2, 2 * _DIM), jnp.float32),
    )(tt)


def _make_sc_kernel():
    mesh = plsc.VectorSubcoreMesh(core_axis_name="c", subcore_axis_name="s")

    @functools.partial(
        pl.kernel,
        mesh=mesh,
        compiler_params=pltpu.CompilerParams(
            use_tc_tiling_on_sc=False, needs_layout_passes=False),
        out_type=jax.ShapeDtypeStruct((_HIST, _BATCH, _DIM), jnp.float32),
        scratch_types=[
            pltpu.VMEM((_PER_W,), jnp.int32),
            pltpu.VMEM((_PER_W,), jnp.int32),
        ] + [pltpu.VMEM((_HB, _DIM), jnp.float32)] * _NBUF
          + [
            pltpu.SemaphoreType.DMA,
            pltpu.SemaphoreType.DMA,
        ] + [pltpu.SemaphoreType.DMA] * _NBUF
          + [pltpu.SemaphoreType.DMA] * _NBUF,
    )
    def tok_emb(xf_hbm, table_hbm, idmap_hbm, out_hbm, xv, mv, *bufs):
        rows = bufs[:_NBUF]
        sem_x = bufs[_NBUF]
        sem_m = bufs[_NBUF + 1]
        sem_r = bufs[_NBUF + 2:2 * _NBUF + 2]
        sem_o = bufs[2 * _NBUF + 2:]
        wid = lax.axis_index("s") * 2 + lax.axis_index("c")
        col0 = wid * _HB

        def x_copy(h):
            return pltpu.make_async_copy(
                xf_hbm.at[pl.ds(h * _BATCH + col0, _HB)],
                xv.at[pl.ds(h * _HB, _HB)], sem_x)

        def g_copy(h, s):
            return pltpu.make_async_copy(
                table_hbm.at[mv.at[pl.ds(h * _HB, _HB)]], rows[s], sem_r[s])

        def o_copy(h, s):
            return pltpu.make_async_copy(
                rows[s], out_hbm.at[h, pl.ds(col0, _HB), :], sem_o[s])

        for h in range(_HIST):
            x_copy(h).start()
        for h in range(_HIST):
            x_copy(h).wait()
        pltpu.make_async_copy(idmap_hbm.at[xv], mv, sem_m).start()
        pltpu.make_async_copy(idmap_hbm.at[xv], mv, sem_m).wait()

        for g in range(_PER_W // 16):
            sl = pl.ds(g * 16, 16)
            v = mv[sl]
            mv[sl] = ((v & -_TBLK) | ((v & (_TBLK // 2 - 1)) << 1)
                      | ((v >> _TBLK_HALF_LOG2) & 1))

        for h in range(_HIST + _SKEW):
            if h < _HIST:
                s = h % _NBUF
                if h >= _NBUF:
                    o_copy(h - _NBUF, s).wait()
                g_copy(h, s).start()
            j = h - _SKEW
            if 0 <= j < _HIST:
                sj = j % _NBUF
                g_copy(j, sj).wait()
                o_copy(j, sj).start()

        for j in range(_HIST - _NBUF, _HIST):
            o_copy(j, j % _NBUF).wait()

    return tok_emb


_SC_KERNEL = _make_sc_kernel()


def kernel(x, table, id_mapper):
    xf = _flatten_x(jnp.transpose(x))
    tf = _flatten_table(jnp.transpose(table))
    table2 = tf.reshape(_TROWS, _DIM)
    out = _SC_KERNEL(xf, table2, id_mapper)
    return jnp.transpose(out, (1, 0, 2))
